# baseline (speedup 1.0000x reference)
import jax
import jax.numpy as jnp
import numpy as np
from jax import lax
from jax.experimental import pallas as pl

_B = 16384
_R = 128
_C = 128
_G = 4
_BR = _R // _G

_KS1 = 42
_KS2 = (42 ^ 0x1BD11BDA) & 0xFFFFFFFF

_ROT_A = (13, 15, 26, 6)
_ROT_B = (17, 29, 16, 24)


def _u32(v):
    return np.uint32(v & 0xFFFFFFFF)


def _rotl(v, r):
    return lax.shift_left(v, np.uint32(r)) | lax.shift_right_logical(
        v, np.uint32(32 - r))


def _round(x0, x1, r):
    x0 = x0 + x1
    x1 = x0 ^ _rotl(x1, r)
    return x0, x1


def _bits_from_x1(x1):
    x0 = x1
    x1 = x0 ^ _rotl(x1, _ROT_A[0])
    for r in _ROT_A[1:]:
        x0, x1 = _round(x0, x1, r)
    x0 = x0 + _u32(_KS1)
    x1 = x1 + _u32(_KS2 + 1)
    for r in _ROT_B:
        x0, x1 = _round(x0, x1, r)
    x0 = x0 + _u32(_KS2)
    x1 = x1 + _u32(0 + 2)
    for r in _ROT_A:
        x0, x1 = _round(x0, x1, r)
    x1 = x1 + _u32(_KS1 + 3)
    for r in _ROT_B:
        x0, x1 = _round(x0, x1, r)
    x0 = x0 + _u32(_KS1)
    x1 = x1 + _u32(_KS2 + 4)
    for r in _ROT_A:
        x0, x1 = _round(x0, x1, r)
    x0 = x0 + _u32(_KS2)
    x1 = x1 + _u32(0 + 5)
    return x0 ^ x1


def _gumbel_from_bits(bits):
    float_bits = lax.shift_right_logical(bits, np.uint32(9)) | np.uint32(
        0x3F800000)
    floats = lax.bitcast_convert_type(float_bits, jnp.float32) - jnp.float32(1.0)
    tiny = jnp.float32(np.finfo(np.float32).tiny)
    u = lax.max(tiny, floats * (jnp.float32(1.0) - tiny) + tiny)
    return -jnp.log(-jnp.log(u))


def _body(x_ref, act_ref, ent_ref, lpa_ref):
    p = pl.program_id(0)
    x = x_ref[...]
    r = lax.broadcasted_iota(jnp.uint32, (_BR, _C), 0)
    c = lax.broadcasted_iota(jnp.uint32, (_BR, _C), 1)
    base = (p.astype(jnp.uint32) * np.uint32(_BR) + r) * np.uint32(2 * _C)
    f0_42 = base + c * np.uint32(2) + np.uint32(42)
    g0 = _gumbel_from_bits(_bits_from_x1(f0_42))
    g1 = _gumbel_from_bits(_bits_from_x1(f0_42 + np.uint32(1)))

    s = jax.nn.sigmoid(x)
    comp = jnp.float32(1.0) - s
    lp0 = jnp.log(comp)
    lp1 = jnp.log(s)
    ent_ref[...] = -(lp0 * comp + lp1 * s)
    take1 = (g1 + lp1) > (g0 + lp0)
    act_ref[...] = take1.astype(jnp.int32)
    lpa_ref[...] = jnp.where(take1, lp1, lp0)


def kernel(x):
    x2 = x.reshape(_R, _C)
    spec = pl.BlockSpec((_BR, _C), lambda p: (p, 0))
    act, ent, lpa = pl.pallas_call(
        _body,
        grid=(_G,),
        in_specs=[spec],
        out_specs=(spec, spec, spec),
        out_shape=(
            jax.ShapeDtypeStruct((_R, _C), jnp.int32),
            jax.ShapeDtypeStruct((_R, _C), jnp.float32),
            jax.ShapeDtypeStruct((_R, _C), jnp.float32),
        ),
    )(x2)
    return (act.reshape(_B, 1), ent.reshape(_B, 1), lpa.reshape(_B, 1))


# speedup vs baseline: 1.4011x; 1.4011x over previous
import jax
import jax.numpy as jnp
import numpy as np
from jax import lax
from jax.experimental import pallas as pl

_B = 16384
_R = 128
_C = 128
_G = 4
_BR = _R // _G

_KS1 = 42
_KS2 = (42 ^ 0x1BD11BDA) & 0xFFFFFFFF

_ROT_A = (13, 15, 26, 6)
_ROT_B = (17, 29, 16, 24)


def _u32(v):
    return np.uint32(v & 0xFFFFFFFF)


def _rotl(v, r):
    return lax.shift_left(v, np.uint32(r)) | lax.shift_right_logical(
        v, np.uint32(32 - r))


def _round(x0, x1, r):
    x0 = x0 + x1
    x1 = x0 ^ _rotl(x1, r)
    return x0, x1


def _bits_from_x1(x1):
    x0 = x1
    x1 = x0 ^ _rotl(x1, _ROT_A[0])
    for r in _ROT_A[1:]:
        x0, x1 = _round(x0, x1, r)
    x0 = x0 + _u32(_KS1)
    x1 = x1 + _u32(_KS2 + 1)
    for r in _ROT_B:
        x0, x1 = _round(x0, x1, r)
    x0 = x0 + _u32(_KS2)
    x1 = x1 + _u32(0 + 2)
    for r in _ROT_A:
        x0, x1 = _round(x0, x1, r)
    x1 = x1 + _u32(_KS1 + 3)
    for r in _ROT_B:
        x0, x1 = _round(x0, x1, r)
    x0 = x0 + _u32(_KS1)
    x1 = x1 + _u32(_KS2 + 4)
    for r in _ROT_A:
        x0, x1 = _round(x0, x1, r)
    x0 = x0 + _u32(_KS2)
    x1 = x1 + _u32(0 + 5)
    return x0 ^ x1


def _gumbel_from_bits(bits):
    float_bits = lax.shift_right_logical(bits, np.uint32(9)) | np.uint32(
        0x3F800000)
    floats = lax.bitcast_convert_type(float_bits, jnp.float32) - jnp.float32(1.0)
    tiny = jnp.float32(np.finfo(np.float32).tiny)
    u = lax.max(tiny, floats * (jnp.float32(1.0) - tiny) + tiny)
    return -jnp.log(-jnp.log(u))


def _body(x_ref, act_ref, ent_ref, lpa_ref):
    x = x_ref[...]
    r = lax.broadcasted_iota(jnp.uint32, (_R, _C), 0)
    c = lax.broadcasted_iota(jnp.uint32, (_R, _C), 1)
    f0_42 = r * np.uint32(2 * _C) + c * np.uint32(2) + np.uint32(42)
    g0 = _gumbel_from_bits(_bits_from_x1(f0_42))
    g1 = _gumbel_from_bits(_bits_from_x1(f0_42 + np.uint32(1)))

    s = jax.nn.sigmoid(x)
    comp = jnp.float32(1.0) - s
    lp0 = jnp.log(comp)
    lp1 = jnp.log(s)
    ent_ref[...] = -(lp0 * comp + lp1 * s)
    take1 = (g1 + lp1) > (g0 + lp0)
    act_ref[...] = take1.astype(jnp.int32)
    lpa_ref[...] = jnp.where(take1, lp1, lp0)


def kernel(x):
    x2 = x.reshape(_R, _C)
    act, ent, lpa = pl.pallas_call(
        _body,
        out_shape=(
            jax.ShapeDtypeStruct((_R, _C), jnp.int32),
            jax.ShapeDtypeStruct((_R, _C), jnp.float32),
            jax.ShapeDtypeStruct((_R, _C), jnp.float32),
        ),
    )(x2)
    return (act.reshape(_B, 1), ent.reshape(_B, 1), lpa.reshape(_B, 1))
